# baseline (speedup 1.0000x reference)
import functools

import jax
import jax.numpy as jnp
from jax import lax
from jax.experimental import pallas as pl
from jax.experimental.pallas import tpu as pltpu
from jax.experimental.pallas import tpu_sc as plsc

_B, _L, _D = 4096, 200, 64
_NW = 32
_NC = 256
_TN = _NC // 128
_UNITS_N = _B // _NC
_UNITS = _L * _UNITS_N
_UNITS_W = _UNITS // _NW
_NBUF = 2
_LANE = 16


def _emb_body(idx_hbm, table_hbm, out_hbm, idx_v, g_v, gsem, wsem):
    wid = lax.axis_index("s") * 2 + lax.axis_index("c")
    u0 = wid * _UNITS_W

    def _gather(u, b):
        l = u // _UNITS_N
        c = u % _UNITS_N
        pltpu.sync_copy(idx_hbm.at[l, pl.ds(c * _NC, _NC)], idx_v.at[b])
        for tn in range(_TN):
            pltpu.async_copy(
                table_hbm.at[idx_v.at[b, pl.ds(tn * 128, 128)]],
                g_v.at[b, tn],
                gsem.at[b],
            )

    def _gather_wait(b):
        for tn in range(_TN):
            pltpu.make_async_copy(
                table_hbm.at[idx_v.at[b, pl.ds(tn * 128, 128)]],
                g_v.at[b, tn],
                gsem.at[b],
            ).wait()

    def _write(u, b):
        l = u // _UNITS_N
        c = u % _UNITS_N
        for d in range(_D):
            pltpu.async_copy(
                g_v.at[b, :, :, d],
                out_hbm.at[l, d // 8, pl.ds(_TN * c, _TN), d % 8],
                wsem.at[b],
            )

    def _write_wait(u, b):
        l = u // _UNITS_N
        c = u % _UNITS_N
        for d in range(_D):
            pltpu.make_async_copy(
                g_v.at[b, :, :, d],
                out_hbm.at[l, d // 8, pl.ds(_TN * c, _TN), d % 8],
                wsem.at[b],
            ).wait()

    for b in range(_NBUF):
        _gather(u0 + b, b)

    def round_body(g, carry):
        for b in range(_NBUF):
            u = u0 + g * _NBUF + b
            _gather_wait(b)
            _write(u, b)
            _write_wait(u, b)
            _gather(u + _NBUF, b)
        return carry

    lax.fori_loop(0, _UNITS_W // _NBUF - 1, round_body, 0)

    for b in range(_NBUF):
        u = u0 + (_UNITS_W // _NBUF - 1) * _NBUF + b
        _gather_wait(b)
        _write(u, b)
        _write_wait(u, b)


_emb = functools.partial(
    pl.kernel,
    out_type=jax.ShapeDtypeStruct((_L, _D // 8, _B // 128, 8, 128), jnp.float32),
    mesh=plsc.VectorSubcoreMesh(core_axis_name="c", subcore_axis_name="s"),
    scratch_types=[
        pltpu.VMEM((_NBUF, _NC), jnp.int32),
        pltpu.VMEM((_NBUF, _TN, 128, _D), jnp.float32),
        pltpu.SemaphoreType.DMA((_NBUF,)),
        pltpu.SemaphoreType.DMA((_NBUF,)),
    ],
    compiler_params=pltpu.CompilerParams(
        use_tc_tiling_on_sc=False, needs_layout_passes=False
    ),
)(_emb_body)


@jax.jit
def kernel(token_ids, weight):
    idx2d = token_ids.T
    out5 = _emb(idx2d, weight)
    out = jnp.transpose(out5, (2, 4, 0, 1, 3))
    return out.reshape(_B, _L, _D)


# speedup vs baseline: 60.6306x; 60.6306x over previous
import functools

import jax
import jax.numpy as jnp
from jax import lax
from jax.experimental import pallas as pl
from jax.experimental.pallas import tpu as pltpu
from jax.experimental.pallas import tpu_sc as plsc

_B, _L, _D = 4096, 200, 64
_NW = 32
_NC = 256
_TN = _NC // 128
_UNITS_N = _B // _NC
_UNITS = _L * _UNITS_N
_UNITS_W = _UNITS // _NW
_NBUF = 2
_LANE = 16


def _emb_body(idx_hbm, table_hbm, out_hbm, idx_v, g_v, t_v, gsem, wsem):
    wid = lax.axis_index("s") * 2 + lax.axis_index("c")
    u0 = wid * _UNITS_W
    iota = lax.iota(jnp.int32, _LANE)
    rows_j = [iota + (16 * j) for j in range(_NC // _LANE)]

    def _gather(u, b):
        l = u // _UNITS_N
        c = u % _UNITS_N
        pltpu.sync_copy(idx_hbm.at[l, pl.ds(c * _NC, _NC)], idx_v.at[b])
        pltpu.async_copy(table_hbm.at[idx_v.at[b]], g_v.at[b], gsem.at[b])

    def _gather_wait(b):
        pltpu.make_async_copy(
            table_hbm.at[idx_v.at[b]], g_v.at[b], gsem.at[b]
        ).wait()

    def _transpose(b):
        g2 = g_v.at[b]

        def d_body(d, carry):
            a = d // 8
            bb = d % 8
            dvec = jnp.full((_LANE,), 0, jnp.int32) + d
            for j in range(_NC // _LANE):
                v = plsc.load_gather(g2, [rows_j[j], dvec])
                t_v[b, a, j // 8, bb, pl.ds((j % 8) * _LANE, _LANE)] = v
            return carry

        lax.fori_loop(0, _D, d_body, 0)

    def _write(u, b):
        l = u // _UNITS_N
        c = u % _UNITS_N
        pltpu.async_copy(
            t_v.at[b], out_hbm.at[l, :, pl.ds(_TN * c, _TN)], wsem.at[b]
        )

    def _write_wait(u, b):
        l = u // _UNITS_N
        c = u % _UNITS_N
        pltpu.make_async_copy(
            t_v.at[b], out_hbm.at[l, :, pl.ds(_TN * c, _TN)], wsem.at[b]
        ).wait()

    for b in range(_NBUF):
        _gather(u0 + b, b)

    def round_body(g, carry):
        for b in range(_NBUF):
            u = u0 + g * _NBUF + b
            _gather_wait(b)
            _transpose(b)
            _write(u, b)
            _write_wait(u, b)
            _gather(u + _NBUF, b)
        return carry

    lax.fori_loop(0, _UNITS_W // _NBUF - 1, round_body, 0)

    for b in range(_NBUF):
        u = u0 + (_UNITS_W // _NBUF - 1) * _NBUF + b
        _gather_wait(b)
        _transpose(b)
        _write(u, b)
        _write_wait(u, b)


_emb = functools.partial(
    pl.kernel,
    out_type=jax.ShapeDtypeStruct((_L, _D // 8, _B // 128, 8, 128), jnp.float32),
    mesh=plsc.VectorSubcoreMesh(core_axis_name="c", subcore_axis_name="s"),
    scratch_types=[
        pltpu.VMEM((_NBUF, _NC), jnp.int32),
        pltpu.VMEM((_NBUF, _NC, _D), jnp.float32),
        pltpu.VMEM((_NBUF, _D // 8, _TN, 8, 128), jnp.float32),
        pltpu.SemaphoreType.DMA((_NBUF,)),
        pltpu.SemaphoreType.DMA((_NBUF,)),
    ],
    compiler_params=pltpu.CompilerParams(
        use_tc_tiling_on_sc=False, needs_layout_passes=False
    ),
)(_emb_body)


@jax.jit
def kernel(token_ids, weight):
    idx2d = token_ids.T
    out5 = _emb(idx2d, weight)
    out = jnp.transpose(out5, (2, 4, 0, 1, 3))
    return out.reshape(_B, _L, _D)


# speedup vs baseline: 127.3966x; 2.1012x over previous
import functools

import jax
import jax.numpy as jnp
from jax import lax
from jax.experimental import pallas as pl
from jax.experimental.pallas import tpu as pltpu
from jax.experimental.pallas import tpu_sc as plsc

_B, _L, _D = 4096, 200, 64
_N = _B * _L
_NW = 32
_PER_W = _N // _NW
_CHUNK = 320
_NBUF = 4
_NCHUNK = _PER_W // _CHUNK
_NROUND = _NCHUNK // _NBUF


def _emb_body(idx_hbm, table_hbm, out_hbm, idx_v, rows_v, gsem, wsem):
    wid = lax.axis_index("s") * 2 + lax.axis_index("c")
    base = wid * _PER_W
    pltpu.sync_copy(idx_hbm.at[pl.ds(base, _PER_W)], idx_v)

    def _gather_args(c, b):
        off = pl.multiple_of(c * _CHUNK, _CHUNK)
        return (
            table_hbm.at[idx_v.at[pl.ds(off, _CHUNK)]],
            rows_v.at[b],
            gsem.at[b],
        )

    def _write_args(c, b):
        off = pl.multiple_of(c * _CHUNK, _CHUNK)
        return (
            rows_v.at[b],
            out_hbm.at[pl.ds(base + off, _CHUNK), pl.ds(0, _D)],
            wsem.at[b],
        )

    for b in range(_NBUF):
        pltpu.async_copy(*_gather_args(b, b))

    def round_body(g, carry):
        for b in range(_NBUF):
            c = g * _NBUF + b
            pltpu.make_async_copy(*_gather_args(c, b)).wait()
            pltpu.async_copy(*_write_args(c, b))
            pltpu.make_async_copy(*_write_args(c, b)).wait()
            pltpu.async_copy(*_gather_args(c + _NBUF, b))
        return carry

    lax.fori_loop(0, _NROUND - 1, round_body, 0)

    for b in range(_NBUF):
        c = (_NROUND - 1) * _NBUF + b
        pltpu.make_async_copy(*_gather_args(c, b)).wait()
        pltpu.async_copy(*_write_args(c, b))
        pltpu.make_async_copy(*_write_args(c, b)).wait()


_emb = functools.partial(
    pl.kernel,
    out_type=jax.ShapeDtypeStruct((_N, 2 * _D), jnp.float32),
    mesh=plsc.VectorSubcoreMesh(core_axis_name="c", subcore_axis_name="s"),
    scratch_types=[
        pltpu.VMEM((_PER_W,), jnp.int32),
        pltpu.VMEM((_NBUF, _CHUNK, _D), jnp.float32),
        pltpu.SemaphoreType.DMA((_NBUF,)),
        pltpu.SemaphoreType.DMA((_NBUF,)),
    ],
    compiler_params=pltpu.CompilerParams(
        use_tc_tiling_on_sc=False, needs_layout_passes=False
    ),
)(_emb_body)


@jax.jit
def kernel(token_ids, weight):
    idx = token_ids.reshape(_N).astype(jnp.int32)
    out2 = _emb(idx, weight)
    return out2[:, : _D].reshape(_B, _L, _D)
